# baseline (speedup 1.0000x reference)
import functools

import jax
import jax.numpy as jnp
from jax import lax
from jax.experimental import pallas as pl
from jax.experimental.pallas import tpu as pltpu
from jax.experimental.pallas import tpu_sc as plsc

NUM_E = 8192
DIM = 256
M = 4096
BM = 256
BN = 512
N_TILES = NUM_E // BN
M_BLOCKS = M // BM


LC = 128
RS = 64


def _vq_tc_body(x_ref, nemb_ref, colidx_ref, onehot_ref, idx_ref,
                dmin_ref, scores_ref, esq_ref):
    x = x_ref[...]
    xsq = jnp.sum(x * x, axis=1, keepdims=True)
    first = pl.program_id(0) == 0
    for j in range(N_TILES):
        e = nemb_ref[j * BN:(j + 1) * BN, :]
        scores_ref[:, j * BN:(j + 1) * BN] = lax.dot_general(
            x, e, (((1,), (1,)), ((), ())),
            preferred_element_type=jnp.float32)

        @pl.when(first)
        def _esq_tile():
            esq_ref[0, j * BN:(j + 1) * BN] = jnp.sum(e * e, axis=1)
    best_idx_parts = []
    best_val_parts = []
    for r in range(BM // RS):
        xsq_r = xsq[r * RS:(r + 1) * RS, :]
        bv = jnp.full((RS, LC), jnp.inf, dtype=jnp.float32)
        bi = jnp.zeros((RS, LC), dtype=jnp.int32)
        for c in range(NUM_E // LC):
            m = scores_ref[r * RS:(r + 1) * RS, c * LC:(c + 1) * LC]
            d = (xsq_r - 2.0 * m) + esq_ref[0, c * LC:(c + 1) * LC][None, :]
            pred = d < bv
            bi = jnp.where(pred, colidx_ref[0, c * LC:(c + 1) * LC][None, :], bi)
            bv = jnp.where(pred, d, bv)
        bvr = jnp.min(bv, axis=1)
        bir = jnp.min(jnp.where(bv == bvr[:, None], bi, NUM_E), axis=1)
        best_val_parts.append(bvr)
        best_idx_parts.append(bir)
    best_val = jnp.concatenate(best_val_parts)
    best_idx = jnp.concatenate(best_idx_parts)
    idx_ref[0, 0, :] = best_idx
    dmin_ref[0, 0, :] = best_val
    col = lax.broadcasted_iota(jnp.int32, (BM, BN), 1)
    for j in range(N_TILES):
        shifted = best_idx - j * BN
        onehot_ref[:, j * BN:(j + 1) * BN] = (
            shifted[:, None] == col).astype(jnp.float32)


def _vq_tc(flat, nemb, colidx):
    return pl.pallas_call(
        _vq_tc_body,
        grid=(M_BLOCKS,),
        in_specs=[
            pl.BlockSpec((BM, DIM), lambda i: (i, 0)),
            pl.BlockSpec((NUM_E, DIM), lambda i: (0, 0)),
            pl.BlockSpec((1, NUM_E), lambda i: (0, 0)),
        ],
        out_specs=[
            pl.BlockSpec((BM, NUM_E), lambda i: (i, 0)),
            pl.BlockSpec((1, 1, BM), lambda i: (i, 0, 0)),
            pl.BlockSpec((1, 1, BM), lambda i: (i, 0, 0)),
        ],
        out_shape=[
            jax.ShapeDtypeStruct((M, NUM_E), jnp.float32),
            jax.ShapeDtypeStruct((M_BLOCKS, 1, BM), jnp.int32),
            jax.ShapeDtypeStruct((M_BLOCKS, 1, BM), jnp.float32),
        ],
        scratch_shapes=[pltpu.VMEM((BM, NUM_E), jnp.float32),
                        pltpu.VMEM((1, NUM_E), jnp.float32)],
    )(flat, nemb, colidx)


_NC, _NS = 2, 16
_NW = _NC * _NS
_BPW = M // _NW
_UPW = M // _NS


_FS = NUM_E // _NS


def _sc_body(table_hbm, idx_hbm, out_hbm, cnt_hbm,
             idx_v, rows_v, ones_v, idx2a_v, idx2b_v,
             flags_sh, cnt_sh, flags_v, cnt_v, sem):
    c = lax.axis_index("c")
    s = lax.axis_index("s")
    wid = s * _NC + c
    base = wid * _BPW
    pltpu.sync_copy(idx_hbm.at[pl.ds(base, _BPW)], idx_v)
    pltpu.async_copy(table_hbm.at[idx_v], rows_v, sem).wait()
    pltpu.sync_copy(rows_v, out_hbm.at[pl.ds(base, _BPW)])

    @pl.when(c == 0)
    def _zero_flags():
        for i in range(_FS // 16):
            flags_v[pl.ds(i * 16, 16)] = jnp.zeros((16,), jnp.float32)
        pltpu.sync_copy(flags_v, flags_sh.at[pl.ds(s * _FS, _FS)])

    plsc.subcore_barrier()

    @pl.when(c == 0)
    def _scatter_ones():
        for i in range(_BPW // 16):
            ones_v[pl.ds(i * 16, 16)] = jnp.ones((16,), jnp.float32)
        ubase = s * _UPW
        pltpu.sync_copy(idx_hbm.at[pl.ds(ubase, _BPW)], idx2a_v)
        pltpu.sync_copy(idx_hbm.at[pl.ds(ubase + _BPW, _BPW)], idx2b_v)
        pltpu.sync_copy(ones_v, flags_sh.at[idx2a_v], add=True)
        pltpu.sync_copy(ones_v, flags_sh.at[idx2b_v], add=True)

    plsc.subcore_barrier()

    @pl.when(c == 0)
    def _count_slice():
        pltpu.sync_copy(flags_sh.at[pl.ds(s * _FS, _FS)], flags_v)
        def cstep(i, acc):
            f = flags_v[pl.ds(i * 16, 16)]
            return acc + jnp.where(f > 0.0,
                                   jnp.ones((16,), jnp.int32),
                                   jnp.zeros((16,), jnp.int32))
        acc = lax.fori_loop(0, _FS // 16, cstep,
                            jnp.zeros((16,), jnp.int32))
        cnt_v[...] = acc
        pltpu.sync_copy(cnt_v, cnt_sh.at[s])

    plsc.subcore_barrier()

    @pl.when(jnp.logical_and(c == 0, s == 0))
    def _sum_partials():
        total = jnp.zeros((16,), jnp.int32)
        for t in range(_NS):
            pltpu.sync_copy(cnt_sh.at[t], cnt_v)
            total = total + cnt_v[...]
        cnt_v[...] = total
        pltpu.sync_copy(cnt_v, cnt_hbm)


@functools.cache
def _sc_gather_count():
    @functools.partial(
        pl.kernel,
        mesh=plsc.VectorSubcoreMesh(core_axis_name="c", subcore_axis_name="s"),
        out_type=[
            jax.ShapeDtypeStruct((M, DIM), jnp.float32),
            jax.ShapeDtypeStruct((16,), jnp.int32),
        ],
        scratch_types=[
            pltpu.VMEM((_BPW,), jnp.int32),
            pltpu.VMEM((_BPW, DIM), jnp.float32),
            pltpu.VMEM((_BPW,), jnp.float32),
            pltpu.VMEM((_BPW,), jnp.int32),
            pltpu.VMEM((_BPW,), jnp.int32),
            pltpu.VMEM_SHARED((NUM_E,), jnp.float32),
            pltpu.VMEM_SHARED((_NS, 16), jnp.int32),
            pltpu.VMEM((_FS,), jnp.float32),
            pltpu.VMEM((16,), jnp.int32),
            pltpu.SemaphoreType.DMA,
        ],
    )
    def sc_kernel(table_hbm, idx_hbm, out_hbm, cnt_hbm, *scratch):
        _sc_body(table_hbm, idx_hbm, out_hbm, cnt_hbm, *scratch)

    return sc_kernel


def kernel(x, current_cost, embeddings):
    input_shape = x.shape
    flat = x.reshape(-1, DIM)
    sq_norm = jnp.sum(embeddings ** 2, axis=1, keepdims=True)
    nemb = embeddings * lax.rsqrt(jnp.maximum(sq_norm, 1e-12))

    colidx = jnp.arange(NUM_E, dtype=jnp.int32).reshape(1, NUM_E)
    onehot, idx3, dmin3 = _vq_tc(flat, nemb, colidx)
    idx = idx3.reshape(M)

    quant_flat, cnt16 = _sc_gather_count()(nemb, idx)

    quantized = quant_flat.reshape(input_shape)
    mse = jnp.sum(dmin3) / jnp.float32(flat.size)
    loss = mse + current_cost * mse
    unique_codes = jnp.sum(cnt16).astype(jnp.int32)
    encodings_3d = onehot.reshape(input_shape[:-1] + (NUM_E,))
    idx_flat = idx.reshape(input_shape[:-1])
    return (quantized, loss, unique_codes, encodings_3d, idx_flat)


# speedup vs baseline: 1.6728x; 1.6728x over previous
import functools

import jax
import jax.numpy as jnp
from jax import lax
from jax.experimental import pallas as pl
from jax.experimental.pallas import tpu as pltpu
from jax.experimental.pallas import tpu_sc as plsc

NUM_E = 8192
DIM = 256
M = 4096
BM = 256
BN = 512
N_TILES = NUM_E // BN
M_BLOCKS = M // BM


LC = 128
RS = 64


def _vq_tc_body(x_ref, nemb_ref, esq_ref, colidx_ref, onehot_ref, idx_ref,
                dmin_ref, scores_ref):
    x = x_ref[...]
    xsq = jnp.sum(x * x, axis=1, keepdims=True)
    for j in range(N_TILES):
        e = nemb_ref[j * BN:(j + 1) * BN, :]
        scores_ref[:, j * BN:(j + 1) * BN] = lax.dot_general(
            x, e, (((1,), (1,)), ((), ())),
            preferred_element_type=jnp.float32)
    best_idx_parts = []
    best_val_parts = []
    for r in range(BM // RS):
        xsq_r = xsq[r * RS:(r + 1) * RS, :]
        bv = jnp.full((RS, LC), jnp.inf, dtype=jnp.float32)
        bi = jnp.zeros((RS, LC), dtype=jnp.int32)
        for c in range(NUM_E // LC):
            m = scores_ref[r * RS:(r + 1) * RS, c * LC:(c + 1) * LC]
            d = (xsq_r - 2.0 * m) + esq_ref[0, c * LC:(c + 1) * LC][None, :]
            pred = d < bv
            bi = jnp.where(pred, colidx_ref[0, c * LC:(c + 1) * LC][None, :], bi)
            bv = jnp.where(pred, d, bv)
        bvr = jnp.min(bv, axis=1)
        bir = jnp.min(jnp.where(bv == bvr[:, None], bi, NUM_E), axis=1)
        best_val_parts.append(bvr)
        best_idx_parts.append(bir)
    best_val = jnp.concatenate(best_val_parts)
    best_idx = jnp.concatenate(best_idx_parts)
    idx_ref[0, 0, :] = best_idx
    dmin_ref[0, 0, :] = best_val
    col = lax.broadcasted_iota(jnp.int32, (BM, BN), 1)
    for j in range(N_TILES):
        shifted = best_idx - j * BN
        onehot_ref[:, j * BN:(j + 1) * BN] = (
            shifted[:, None] == col).astype(jnp.float32)


def _vq_tc(flat, nemb, esq, colidx):
    return pl.pallas_call(
        _vq_tc_body,
        grid=(M_BLOCKS,),
        in_specs=[
            pl.BlockSpec((BM, DIM), lambda i: (i, 0)),
            pl.BlockSpec((NUM_E, DIM), lambda i: (0, 0)),
            pl.BlockSpec((1, NUM_E), lambda i: (0, 0)),
            pl.BlockSpec((1, NUM_E), lambda i: (0, 0)),
        ],
        out_specs=[
            pl.BlockSpec((BM, NUM_E), lambda i: (i, 0)),
            pl.BlockSpec((1, 1, BM), lambda i: (i, 0, 0)),
            pl.BlockSpec((1, 1, BM), lambda i: (i, 0, 0)),
        ],
        out_shape=[
            jax.ShapeDtypeStruct((M, NUM_E), jnp.float32),
            jax.ShapeDtypeStruct((M_BLOCKS, 1, BM), jnp.int32),
            jax.ShapeDtypeStruct((M_BLOCKS, 1, BM), jnp.float32),
        ],
        scratch_shapes=[pltpu.VMEM((BM, NUM_E), jnp.float32)],
    )(flat, nemb, esq, colidx)


_NC, _NS = 2, 16
_NW = _NC * _NS
_BPW = M // _NW
_UPW = M // _NS


_FS = NUM_E // _NS


def _sc_body(table_hbm, idx_hbm, out_hbm, cnt_hbm,
             idx_v, rows_v, ones_v, idx2a_v, idx2b_v,
             flags_sh, cnt_sh, flags_v, cnt_v, sem):
    c = lax.axis_index("c")
    s = lax.axis_index("s")
    wid = s * _NC + c
    base = wid * _BPW
    pltpu.sync_copy(idx_hbm.at[pl.ds(base, _BPW)], idx_v)
    pltpu.async_copy(table_hbm.at[idx_v], rows_v, sem).wait()
    pltpu.sync_copy(rows_v, out_hbm.at[pl.ds(base, _BPW)])

    @pl.when(c == 0)
    def _zero_flags():
        for i in range(_FS // 16):
            flags_v[pl.ds(i * 16, 16)] = jnp.zeros((16,), jnp.float32)
        pltpu.sync_copy(flags_v, flags_sh.at[pl.ds(s * _FS, _FS)])

    plsc.subcore_barrier()

    @pl.when(c == 0)
    def _scatter_ones():
        for i in range(_BPW // 16):
            ones_v[pl.ds(i * 16, 16)] = jnp.ones((16,), jnp.float32)
        ubase = s * _UPW
        pltpu.sync_copy(idx_hbm.at[pl.ds(ubase, _BPW)], idx2a_v)
        pltpu.sync_copy(idx_hbm.at[pl.ds(ubase + _BPW, _BPW)], idx2b_v)
        pltpu.sync_copy(ones_v, flags_sh.at[idx2a_v], add=True)
        pltpu.sync_copy(ones_v, flags_sh.at[idx2b_v], add=True)

    plsc.subcore_barrier()

    @pl.when(c == 0)
    def _count_slice():
        pltpu.sync_copy(flags_sh.at[pl.ds(s * _FS, _FS)], flags_v)
        def cstep(i, acc):
            f = flags_v[pl.ds(i * 16, 16)]
            return acc + jnp.where(f > 0.0,
                                   jnp.ones((16,), jnp.int32),
                                   jnp.zeros((16,), jnp.int32))
        acc = lax.fori_loop(0, _FS // 16, cstep,
                            jnp.zeros((16,), jnp.int32))
        cnt_v[...] = acc
        pltpu.sync_copy(cnt_v, cnt_sh.at[s])

    plsc.subcore_barrier()

    @pl.when(jnp.logical_and(c == 0, s == 0))
    def _sum_partials():
        total = jnp.zeros((16,), jnp.int32)
        for t in range(_NS):
            pltpu.sync_copy(cnt_sh.at[t], cnt_v)
            total = total + cnt_v[...]
        cnt_v[...] = total
        pltpu.sync_copy(cnt_v, cnt_hbm)


@functools.cache
def _sc_gather_count():
    @functools.partial(
        pl.kernel,
        mesh=plsc.VectorSubcoreMesh(core_axis_name="c", subcore_axis_name="s"),
        out_type=[
            jax.ShapeDtypeStruct((M, DIM), jnp.float32),
            jax.ShapeDtypeStruct((16,), jnp.int32),
        ],
        scratch_types=[
            pltpu.VMEM((_BPW,), jnp.int32),
            pltpu.VMEM((_BPW, DIM), jnp.float32),
            pltpu.VMEM((_BPW,), jnp.float32),
            pltpu.VMEM((_BPW,), jnp.int32),
            pltpu.VMEM((_BPW,), jnp.int32),
            pltpu.VMEM_SHARED((NUM_E,), jnp.float32),
            pltpu.VMEM_SHARED((_NS, 16), jnp.int32),
            pltpu.VMEM((_FS,), jnp.float32),
            pltpu.VMEM((16,), jnp.int32),
            pltpu.SemaphoreType.DMA,
        ],
    )
    def sc_kernel(table_hbm, idx_hbm, out_hbm, cnt_hbm, *scratch):
        _sc_body(table_hbm, idx_hbm, out_hbm, cnt_hbm, *scratch)

    return sc_kernel


def kernel(x, current_cost, embeddings):
    input_shape = x.shape
    flat = x.reshape(-1, DIM)
    sq_norm = jnp.sum(embeddings ** 2, axis=1, keepdims=True)
    nemb = embeddings * lax.rsqrt(jnp.maximum(sq_norm, 1e-12))
    esq = jnp.sum(nemb ** 2, axis=1).reshape(1, NUM_E)

    colidx = jnp.arange(NUM_E, dtype=jnp.int32).reshape(1, NUM_E)
    onehot, idx3, dmin3 = _vq_tc(flat, nemb, esq, colidx)
    idx = idx3.reshape(M)

    quant_flat, cnt16 = _sc_gather_count()(nemb, idx)

    quantized = quant_flat.reshape(input_shape)
    mse = jnp.sum(dmin3) / jnp.float32(flat.size)
    loss = mse + current_cost * mse
    unique_codes = jnp.sum(cnt16).astype(jnp.int32)
    encodings_3d = onehot.reshape(input_shape[:-1] + (NUM_E,))
    idx_flat = idx.reshape(input_shape[:-1])
    return (quantized, loss, unique_codes, encodings_3d, idx_flat)
